# Initial kernel scaffold; baseline (speedup 1.0000x reference)
import jax
import jax.numpy as jnp
from jax.experimental import pallas as pl


def kernel(seq, edge_index, edge_weight, W):
    raise NotImplementedError("write your pallas kernel here")


# speedup vs baseline: 3.9410x; 3.9410x over previous
import functools

import jax
import jax.numpy as jnp
from jax import lax
from jax.experimental import pallas as pl
from jax.experimental.pallas import tpu as pltpu
from jax.experimental.pallas import tpu_sc as plsc

N = 10000
E = 320000
D = 128

NUM_CORES = 2
NUM_SUBCORES = 16
NUM_WORKERS = NUM_CORES * NUM_SUBCORES
SUB = 80
NSUB = 4
CHUNK = SUB * NSUB
NCHUNKS = E // CHUNK
ROWS_MAIN = 624
MM_BLOCK = 1000


def _mm_body(x_ref, w_ref, o_ref):
    o_ref[...] = lax.dot_general(
        x_ref[...], w_ref[...], (((1,), (1,)), ((), ())),
        preferred_element_type=jnp.float32)


def _matmul(seq, W):
    return pl.pallas_call(
        _mm_body,
        grid=(N // MM_BLOCK,),
        in_specs=[
            pl.BlockSpec((MM_BLOCK, D), lambda i: (i, 0)),
            pl.BlockSpec((D, D), lambda i: (0, 0)),
        ],
        out_specs=pl.BlockSpec((MM_BLOCK, D), lambda i: (i, 0)),
        out_shape=jax.ShapeDtypeStruct((N, D), jnp.float32),
    )(seq, W)


def _combine_body(a_ref, b_ref, o_ref):
    o_ref[...] = jnp.maximum(a_ref[...] + b_ref[...], 0.0)


def _combine(a, b):
    return pl.pallas_call(
        _combine_body,
        grid=(N // MM_BLOCK,),
        in_specs=[
            pl.BlockSpec((MM_BLOCK, D), lambda i: (i, 0)),
            pl.BlockSpec((MM_BLOCK, D), lambda i: (i, 0)),
        ],
        out_specs=pl.BlockSpec((MM_BLOCK, D), lambda i: (i, 0)),
        out_shape=jax.ShapeDtypeStruct((N, D), jnp.float32),
    )(a, b)


@functools.partial(
    pl.kernel,
    mesh=plsc.VectorSubcoreMesh(core_axis_name="c", subcore_axis_name="s"),
    out_type=jax.ShapeDtypeStruct((NUM_CORES, N, D), jnp.float32),
    scratch_types=[
        pltpu.VMEM((NSUB, SUB), jnp.int32),
        pltpu.VMEM((NSUB, SUB), jnp.int32),
        pltpu.VMEM((CHUNK // 8, D), jnp.float32),
        pltpu.VMEM((CHUNK, D), jnp.float32),
        pltpu.VMEM_SHARED((N, D), jnp.float32),
        pltpu.SemaphoreType.DMA,
    ],
)
def _spmm_sc(h_hbm, col_hbm, row_hbm, w_hbm, part_hbm,
             col_v, row_v, w_v, msg_v, acc, sem):
    c = lax.axis_index("c")
    s = lax.axis_index("s")
    wid = c * NUM_SUBCORES + s

    zero16 = jnp.zeros((16,), jnp.float32)

    def _zero_row(k, carry):
        for j in range(D // 16):
            msg_v[k, pl.ds(j * 16, 16)] = zero16
        return carry

    lax.fori_loop(0, CHUNK, _zero_row, 0)
    r0 = s * ROWS_MAIN

    @pl.when(s == NUM_SUBCORES - 1)
    def _():
        for q in range(2):
            pltpu.sync_copy(msg_v, acc.at[pl.ds(r0 + q * CHUNK, CHUNK)])

    @pl.when(s != NUM_SUBCORES - 1)
    def _():
        for q in range(2):
            pltpu.sync_copy(msg_v.at[pl.ds(0, ROWS_MAIN // 2)],
                            acc.at[pl.ds(r0 + q * (ROWS_MAIN // 2),
                                         ROWS_MAIN // 2)])

    plsc.subcore_barrier()

    start = wid * 31 + jnp.minimum(wid, 8)
    count = jnp.where(wid < 8, 32, 31)

    def _chunk(ci, carry):
        pltpu.sync_copy(col_hbm.at[ci], col_v)
        pltpu.sync_copy(row_hbm.at[ci], row_v)
        pltpu.sync_copy(w_hbm.at[pl.ds(ci * (CHUNK // 8), CHUNK // 8)], w_v)
        for j in range(NSUB):
            pltpu.async_copy(h_hbm.at[col_v.at[j]],
                             msg_v.at[pl.ds(j * SUB, SUB)], sem).wait()

        def _scale(k, inner):
            wk = w_v[k // 8, pl.ds((k % 8) * 16, 16)]
            for j in range(D // 16):
                msg_v[k, pl.ds(j * 16, 16)] = msg_v[k, pl.ds(j * 16, 16)] * wk
            return inner

        lax.fori_loop(0, CHUNK, _scale, 0)
        for j in range(NSUB):
            pltpu.sync_copy(msg_v.at[pl.ds(j * SUB, SUB)],
                            acc.at[row_v.at[j]], add=True)
        return carry

    lax.fori_loop(start, start + count, _chunk, 0)
    plsc.subcore_barrier()

    @pl.when(s == NUM_SUBCORES - 1)
    def _():
        for q in range(2):
            pltpu.sync_copy(acc.at[pl.ds(r0 + q * CHUNK, CHUNK)], msg_v)
            pltpu.sync_copy(msg_v, part_hbm.at[c, pl.ds(r0 + q * CHUNK, CHUNK)])

    @pl.when(s != NUM_SUBCORES - 1)
    def _():
        for q in range(2):
            h0 = r0 + q * (ROWS_MAIN // 2)
            pltpu.sync_copy(acc.at[pl.ds(h0, ROWS_MAIN // 2)],
                            msg_v.at[pl.ds(0, ROWS_MAIN // 2)])
            pltpu.sync_copy(msg_v.at[pl.ds(0, ROWS_MAIN // 2)],
                            part_hbm.at[c, pl.ds(h0, ROWS_MAIN // 2)])


def kernel(seq, edge_index, edge_weight, W):
    col = edge_index[1].astype(jnp.int32).reshape(NCHUNKS, NSUB, SUB)
    row = edge_index[0].astype(jnp.int32).reshape(NCHUNKS, NSUB, SUB)
    wb = jnp.repeat(edge_weight.reshape(E // 8, 8), 16, axis=-1)
    h = _matmul(seq, W)
    part = _spmm_sc(h, col, row, wb)
    return _combine(part[0], part[1])
